# baseline (speedup 1.0000x reference)
import functools

import jax
import jax.numpy as jnp
from jax import lax
from jax.experimental import pallas as pl
from jax.experimental.pallas import tpu as pltpu
from jax.experimental.pallas import tpu_sc as plsc

_NC = 2
_NS = 16
_L = 16

_N = 10000
_E = 320000
_D = 128
_DH = _D // 2

_NP = 10240
_K = 80
_CHUNKS = _E // _NS // _K
_NBUF = 5
_GRP = _CHUNKS // _NBUF
_CCH = _CHUNKS // _NC
_CGRP = _CCH // _NBUF
_RPS = _NP // _NS
_ZR = 128
_CW = _L

_ROW_BLK = 1000


def _dot(a, b):
    return lax.dot_general(a, b, (((1,), (0,)), ((), ())),
                           precision=lax.Precision.HIGHEST,
                           preferred_element_type=jnp.float32)


def _dense_pre(x, W_l, W_r, b):
    def body(x_ref, wl_ref, wr_ref, b_ref, p_ref, r_ref):
        xb = x_ref[...]
        p_ref[...] = _dot(xb, wl_ref[...])
        r_ref[...] = _dot(xb, wr_ref[...]) + b_ref[...]

    n = _N
    grid = (n // _ROW_BLK,)
    row = pl.BlockSpec((_ROW_BLK, _D), lambda i: (i, 0))
    return pl.pallas_call(
        body,
        grid=grid,
        in_specs=[
            row,
            pl.BlockSpec((_D, _D), lambda i: (0, 0)),
            pl.BlockSpec((_D, _D), lambda i: (0, 0)),
            pl.BlockSpec((1, _D), lambda i: (0, 0)),
        ],
        out_specs=[row, row],
        out_shape=[jax.ShapeDtypeStruct((n, _D), jnp.float32)] * 2,
    )(x, W_l, W_r, b.reshape(1, _D))


def _merge_agg(a_ref, c_ref, r):
    cnt = c_ref[0, :, 0:1] + c_ref[1, :, 0:1]
    inv = 1.0 / jnp.maximum(cnt, 1.0)
    return a_ref * inv + r


_agg2 = pl.BlockSpec((_ROW_BLK, _D), lambda i: (i, 0))
_cnt3 = pl.BlockSpec((_NC, _ROW_BLK, _CW), lambda i: (0, i, 0))


def _dense_mid(agg, cnt, r1, W_l, W_r, b):
    def body(a_ref, c_ref, r1_ref, wl_ref, wr_ref, b_ref,
             p_ref, r_ref):
        h = jnp.maximum(_merge_agg(a_ref[...], c_ref[...], r1_ref[...]),
                        0.0)
        p_ref[...] = _dot(h, wl_ref[...])
        r_ref[...] = _dot(h, wr_ref[...]) + b_ref[...]

    n = _N
    grid = (n // _ROW_BLK,)
    row = pl.BlockSpec((_ROW_BLK, _D), lambda i: (i, 0))
    wspec = pl.BlockSpec((_D, _D), lambda i: (0, 0))
    return pl.pallas_call(
        body,
        grid=grid,
        in_specs=[_agg2, _cnt3, row,
                  wspec, wspec, pl.BlockSpec((1, _D), lambda i: (0, 0))],
        out_specs=[row, row],
        out_shape=[jax.ShapeDtypeStruct((n, _D), jnp.float32)] * 2,
    )(agg, cnt, r1, W_l, W_r, b.reshape(1, _D))


def _dense_post(agg, cnt, r2):
    def body(a_ref, c_ref, r2_ref, o_ref):
        o_ref[...] = _merge_agg(a_ref[...], c_ref[...], r2_ref[...])

    n = _N
    grid = (n // _ROW_BLK,)
    row = pl.BlockSpec((_ROW_BLK, _D), lambda i: (i, 0))
    return pl.pallas_call(
        body,
        grid=grid,
        in_specs=[_agg2, _cnt3, row],
        out_specs=row,
        out_shape=jax.ShapeDtypeStruct((n, _D), jnp.float32),
    )(agg, cnt, r2)


_sc_mesh = plsc.VectorSubcoreMesh(core_axis_name="c", subcore_axis_name="s")
_sc_params = pltpu.CompilerParams(use_tc_tiling_on_sc=False)


def _make_seg_sum(with_count):
    out_type = [jax.ShapeDtypeStruct((_NP, _D), jnp.float32)]
    scratch = [
        pltpu.VMEM((_CHUNKS, _K), jnp.int32),
        pltpu.VMEM((_CHUNKS, _K), jnp.int32),
        pltpu.VMEM((_NBUF, _K, _DH), jnp.float32),
        pltpu.VMEM((_ZR, _DH), jnp.float32),
        pltpu.VMEM_SHARED((_NP, _DH), jnp.float32),
    ]
    if with_count:
        out_type = out_type + [
            jax.ShapeDtypeStruct((_NC, _NP, _CW), jnp.float32)]
        scratch = scratch + [
            pltpu.VMEM((_K, _DH), jnp.float32),
        ]
    scratch = scratch + [pltpu.SemaphoreType.DMA] * (2 * _NBUF)

    def body(table_hbm, edges_hbm, *refs):
        if with_count:
            (out, outc, src_v, dst_v, rows_v, zero_v, acc_s,
             ones_v, *sems) = refs
        else:
            out, src_v, dst_v, rows_v, zero_v, acc_s, *sems = refs
        gsem, ssem = sems[:_NBUF], sems[_NBUF:]

        cid = lax.axis_index("c")
        sid = lax.axis_index("s")
        stripe = pl.ds(sid * _RPS, _RPS)

        @pl.loop(0, _ZR)
        def _(i):
            for c in range(_DH // _L):
                zero_v.at[pl.ds(i, 1), pl.ds(c * _L, _L)][...] = (
                    jnp.zeros((1, _L), jnp.float32))

        def zero_stripe():
            for blk in range(_RPS // _ZR):
                base = sid * _RPS + blk * _ZR
                pltpu.sync_copy(zero_v, acc_s.at[pl.ds(base, _ZR), :])

        def wait_gather(table, b):
            pltpu.make_async_copy(
                table.at[src_v.at[b]], rows_v.at[b], gsem[b]).wait()

        def shift_src():
            @pl.loop(0, _CHUNKS)
            def _(i):
                for c in range(_K // _L):
                    sl = src_v.at[pl.ds(i, 1), pl.ds(c * _L, _L)]
                    v = sl[...]
                    sl[...] = v * 2 + cid

        def wait_scatter(b):
            pltpu.make_async_copy(
                rows_v.at[b], acc_s.at[dst_v.at[b]], ssem[b]).wait()

        def data_pass(table):
            for b in range(_NBUF):
                pltpu.async_copy(table.at[src_v.at[b]], rows_v.at[b],
                                 gsem[b])

            @pl.loop(0, _GRP)
            def _(g):
                for b in range(_NBUF):
                    j = g * _NBUF + b
                    wait_gather(table, b)
                    pltpu.async_copy(rows_v.at[b], acc_s.at[dst_v.at[j]],
                                     ssem[b], add=True)

                    @pl.when(g < _GRP - 1)
                    def _():
                        wait_scatter(b)
                        pltpu.async_copy(table.at[src_v.at[j + _NBUF]],
                                         rows_v.at[b], gsem[b])

            for b in range(_NBUF):
                wait_scatter(b)

        def count_pass():
            base = cid * _CCH

            @pl.loop(0, _CGRP)
            def _(g):
                for b in range(_NBUF):
                    j = g * _NBUF + b

                    @pl.when(g > 0)
                    def _():
                        pltpu.make_async_copy(
                            ones_v, acc_s.at[dst_v.at[base + b]],
                            ssem[b]).wait()

                    pltpu.async_copy(ones_v, acc_s.at[dst_v.at[base + j]],
                                     ssem[b], add=True)

            for b in range(_NBUF):
                pltpu.make_async_copy(
                    ones_v, acc_s.at[dst_v.at[base + b]], ssem[b]).wait()

        zero_stripe()
        if with_count:
            @pl.loop(0, _K)
            def _(i):
                for c in range(_DH // _L):
                    ones_v.at[pl.ds(i, 1), pl.ds(c * _L, _L)][...] = (
                        jnp.ones((1, _L), jnp.float32))

        pltpu.sync_copy(edges_hbm.at[0, sid], src_v)
        pltpu.sync_copy(edges_hbm.at[1, sid], dst_v)
        shift_src()
        plsc.subcore_barrier()

        if with_count:
            count_pass()
            plsc.subcore_barrier()
            pltpu.sync_copy(acc_s.at[stripe, pl.ds(0, _CW)],
                            outc.at[cid, stripe, :])
            plsc.subcore_barrier()
            zero_stripe()
            plsc.subcore_barrier()

        data_pass(table_hbm)

        plsc.subcore_barrier()
        pltpu.sync_copy(acc_s.at[stripe, :],
                        out.at[stripe, pl.ds(cid * _DH, _DH)])

    return functools.partial(pl.kernel, mesh=_sc_mesh, out_type=out_type,
                             scratch_types=scratch,
                             compiler_params=_sc_params)(body)


_seg_sum_count = _make_seg_sum(with_count=True)
_seg_sum = _make_seg_sum(with_count=False)


def kernel(x, edge_index, W_l1, b1, W_r1, W_l2, b2, W_r2):
    edges = edge_index.reshape(2, _NS, _CHUNKS, _K)

    p1, r1 = _dense_pre(x, W_l1, W_r1, b1)
    agg1, cnt = _seg_sum_count(p1.reshape(2 * _N, _DH), edges)
    p2, r2 = _dense_mid(agg1, cnt, r1, W_l2, W_r2, b2)
    (agg2,) = _seg_sum(p2.reshape(2 * _N, _DH), edges)
    return _dense_post(agg2, cnt, r2)


# speedup vs baseline: 1.1550x; 1.1550x over previous
import functools

import jax
import jax.numpy as jnp
from jax import lax
from jax.experimental import pallas as pl
from jax.experimental.pallas import tpu as pltpu
from jax.experimental.pallas import tpu_sc as plsc

_NC = 2
_NS = 16
_L = 16

_N = 10000
_E = 320000
_D = 128
_DH = _D // 2

_NP = 10240
_K = 80
_CHUNKS = _E // _NS // _K
_NBUF = 5
_GRP = _CHUNKS // _NBUF
_CCH = _CHUNKS // _NC
_CGRP = _CCH // _NBUF
_RPS = _NP // _NS
_ZR = 128
_CW = _L

_ROW_BLK = 1000


def _dot(a, b):
    return lax.dot_general(a, b, (((1,), (0,)), ((), ())),
                           precision=lax.Precision.HIGHEST,
                           preferred_element_type=jnp.float32)


def _dense_pre(x, W_l, W_r, b):
    def body(x_ref, wl_ref, wr_ref, b_ref, p_ref, r_ref):
        xb = x_ref[...]
        p_ref[...] = _dot(xb, wl_ref[...])
        r_ref[...] = _dot(xb, wr_ref[...]) + b_ref[...]

    n = _N
    grid = (n // _ROW_BLK,)
    row = pl.BlockSpec((_ROW_BLK, _D), lambda i: (i, 0))
    return pl.pallas_call(
        body,
        grid=grid,
        in_specs=[
            row,
            pl.BlockSpec((_D, _D), lambda i: (0, 0)),
            pl.BlockSpec((_D, _D), lambda i: (0, 0)),
            pl.BlockSpec((1, _D), lambda i: (0, 0)),
        ],
        out_specs=[row, row],
        out_shape=[jax.ShapeDtypeStruct((n, _D), jnp.float32)] * 2,
    )(x, W_l, W_r, b.reshape(1, _D))


def _merge_agg(a_ref, c_ref, r):
    cnt = c_ref[0, :, 0:1] + c_ref[1, :, 0:1]
    inv = 1.0 / jnp.maximum(cnt, 1.0)
    return a_ref * inv + r


_agg2 = pl.BlockSpec((_ROW_BLK, _D), lambda i: (i, 0))
_cnt3 = pl.BlockSpec((_NC, _ROW_BLK, _CW), lambda i: (0, i, 0))


def _dense_mid(agg, cnt, r1, W_l, W_r, b):
    def body(a_ref, c_ref, r1_ref, wl_ref, wr_ref, b_ref,
             p_ref, r_ref):
        h = jnp.maximum(_merge_agg(a_ref[...], c_ref[...], r1_ref[...]),
                        0.0)
        p_ref[...] = _dot(h, wl_ref[...])
        r_ref[...] = _dot(h, wr_ref[...]) + b_ref[...]

    n = _N
    grid = (n // _ROW_BLK,)
    row = pl.BlockSpec((_ROW_BLK, _D), lambda i: (i, 0))
    wspec = pl.BlockSpec((_D, _D), lambda i: (0, 0))
    return pl.pallas_call(
        body,
        grid=grid,
        in_specs=[_agg2, _cnt3, row,
                  wspec, wspec, pl.BlockSpec((1, _D), lambda i: (0, 0))],
        out_specs=[row, row],
        out_shape=[jax.ShapeDtypeStruct((n, _D), jnp.float32)] * 2,
    )(agg, cnt, r1, W_l, W_r, b.reshape(1, _D))


def _dense_post(agg, cnt, r2):
    def body(a_ref, c_ref, r2_ref, o_ref):
        o_ref[...] = _merge_agg(a_ref[...], c_ref[...], r2_ref[...])

    n = _N
    grid = (n // _ROW_BLK,)
    row = pl.BlockSpec((_ROW_BLK, _D), lambda i: (i, 0))
    return pl.pallas_call(
        body,
        grid=grid,
        in_specs=[_agg2, _cnt3, row],
        out_specs=row,
        out_shape=jax.ShapeDtypeStruct((n, _D), jnp.float32),
    )(agg, cnt, r2)


_sc_mesh = plsc.VectorSubcoreMesh(core_axis_name="c", subcore_axis_name="s")
_sc_params = pltpu.CompilerParams(use_tc_tiling_on_sc=False)


def _count_kernel():
    out_type = [jax.ShapeDtypeStruct((_NC, _NP, _CW), jnp.float32)]
    scratch = [
        pltpu.VMEM((_CCH, _K), jnp.int32),
        pltpu.VMEM((_K, _CW), jnp.float32),
        pltpu.VMEM((_ZR, _CW), jnp.float32),
        pltpu.VMEM_SHARED((_NP, _CW), jnp.float32),
    ] + [pltpu.SemaphoreType.DMA] * _NBUF

    def body(edges_hbm, outc, dst_v, ones_v, zero_v, acc_s, *sems):
        cid = lax.axis_index("c")
        sid = lax.axis_index("s")
        stripe = pl.ds(sid * _RPS, _RPS)

        @pl.loop(0, _ZR)
        def _(i):
            zero_v.at[pl.ds(i, 1), :][...] = jnp.zeros((1, _CW),
                                                       jnp.float32)

        @pl.loop(0, _K)
        def _(i):
            ones_v.at[pl.ds(i, 1), :][...] = jnp.ones((1, _CW),
                                                      jnp.float32)

        for blk in range(_RPS // _ZR):
            base = sid * _RPS + blk * _ZR
            pltpu.sync_copy(zero_v, acc_s.at[pl.ds(base, _ZR), :])

        pltpu.sync_copy(edges_hbm.at[1, sid, pl.ds(cid * _CCH, _CCH)],
                        dst_v)
        plsc.subcore_barrier()

        @pl.loop(0, _CGRP)
        def _(g):
            for b in range(_NBUF):
                j = g * _NBUF + b

                @pl.when(g > 0)
                def _():
                    pltpu.make_async_copy(
                        ones_v, acc_s.at[dst_v.at[b]], sems[b]).wait()

                pltpu.async_copy(ones_v, acc_s.at[dst_v.at[j]],
                                 sems[b], add=True)

        for b in range(_NBUF):
            pltpu.make_async_copy(
                ones_v, acc_s.at[dst_v.at[b]], sems[b]).wait()

        plsc.subcore_barrier()
        pltpu.sync_copy(acc_s.at[stripe, :], outc.at[cid, stripe, :])

    return functools.partial(pl.kernel, mesh=_sc_mesh, out_type=out_type,
                             scratch_types=scratch,
                             compiler_params=_sc_params)(body)


def _make_seg_sum():
    out_type = [jax.ShapeDtypeStruct((_NP, _D), jnp.float32)]
    scratch = [
        pltpu.VMEM((_CHUNKS, _K), jnp.int32),
        pltpu.VMEM((_CHUNKS, _K), jnp.int32),
        pltpu.VMEM((_NBUF, _K, _DH), jnp.float32),
        pltpu.VMEM((_ZR, _DH), jnp.float32),
        pltpu.VMEM_SHARED((_NP, _DH), jnp.float32),
    ] + [pltpu.SemaphoreType.DMA] * (2 * _NBUF)

    def body(table_hbm, edges_hbm, out, src_v, dst_v, rows_v, zero_v,
             acc_s, *sems):
        gsem, ssem = sems[:_NBUF], sems[_NBUF:]

        cid = lax.axis_index("c")
        sid = lax.axis_index("s")
        stripe = pl.ds(sid * _RPS, _RPS)

        @pl.loop(0, _ZR)
        def _(i):
            for c in range(_DH // _L):
                zero_v.at[pl.ds(i, 1), pl.ds(c * _L, _L)][...] = (
                    jnp.zeros((1, _L), jnp.float32))

        for blk in range(_RPS // _ZR):
            base = sid * _RPS + blk * _ZR
            pltpu.sync_copy(zero_v, acc_s.at[pl.ds(base, _ZR), :])

        pltpu.sync_copy(edges_hbm.at[0, sid], src_v)
        pltpu.sync_copy(edges_hbm.at[1, sid], dst_v)

        @pl.loop(0, _CHUNKS)
        def _(i):
            for c in range(_K // _L):
                sl = src_v.at[pl.ds(i, 1), pl.ds(c * _L, _L)]
                v = sl[...]
                sl[...] = v * 2 + cid

        plsc.subcore_barrier()

        def wait_gather(b):
            pltpu.make_async_copy(
                table_hbm.at[src_v.at[b]], rows_v.at[b], gsem[b]).wait()

        def wait_scatter(b):
            pltpu.make_async_copy(
                rows_v.at[b], acc_s.at[dst_v.at[b]], ssem[b]).wait()

        for b in range(_NBUF):
            pltpu.async_copy(table_hbm.at[src_v.at[b]], rows_v.at[b],
                             gsem[b])

        @pl.loop(0, _GRP)
        def _(g):
            for b in range(_NBUF):
                j = g * _NBUF + b
                wait_gather(b)
                pltpu.async_copy(rows_v.at[b], acc_s.at[dst_v.at[j]],
                                 ssem[b], add=True)

                @pl.when(g < _GRP - 1)
                def _():
                    wait_scatter(b)
                    pltpu.async_copy(table_hbm.at[src_v.at[j + _NBUF]],
                                     rows_v.at[b], gsem[b])

        for b in range(_NBUF):
            wait_scatter(b)

        plsc.subcore_barrier()
        pltpu.sync_copy(acc_s.at[stripe, :],
                        out.at[stripe, pl.ds(cid * _DH, _DH)])

    return functools.partial(pl.kernel, mesh=_sc_mesh, out_type=out_type,
                             scratch_types=scratch,
                             compiler_params=_sc_params)(body)


_count = _count_kernel()
_seg_sum = _make_seg_sum()


def kernel(x, edge_index, W_l1, b1, W_r1, W_l2, b2, W_r2):
    edges = edge_index.reshape(2, _NS, _CHUNKS, _K)

    (cnt,) = _count(edges)
    p1, r1 = _dense_pre(x, W_l1, W_r1, b1)
    (agg1,) = _seg_sum(p1.reshape(2 * _N, _DH), edges)
    p2, r2 = _dense_mid(agg1, cnt, r1, W_l2, W_r2, b2)
    (agg2,) = _seg_sum(p2.reshape(2 * _N, _DH), edges)
    return _dense_post(agg2, cnt, r2)
